# Initial kernel scaffold; baseline (speedup 1.0000x reference)
import jax
import jax.numpy as jnp
from jax.experimental import pallas as pl


def kernel(logits, top_ps, top_ks):
    raise NotImplementedError("write your pallas kernel here")


# speedup vs baseline: 16.4946x; 16.4946x over previous
import jax
import jax.numpy as jnp
from jax import lax
from jax.experimental import pallas as pl

V = 100000
ROWS = 8


def _block(logits_ref, p_ref, k_ref, out_ref):
    x = logits_ref[...]
    p = p_ref[...]
    k = k_ref[...]

    row_max = jnp.max(x, axis=1, keepdims=True)
    probs = jnp.exp(x - row_max)
    z = jnp.sum(probs, axis=1, keepdims=True)
    pz = p * z
    km1 = k - jnp.int32(1)

    bits = lax.bitcast_convert_type(x, jnp.int32)
    key = jnp.where(bits >= 0, bits,
                    jnp.int32(-1) - (bits & jnp.int32(0x7FFFFFFF)))

    def keep(mid):
        gt = key > mid
        c = jnp.sum(gt.astype(jnp.int32), axis=1, keepdims=True)
        s = jnp.sum(jnp.where(gt, probs, 0.0), axis=1, keepdims=True)
        return (c <= km1) & (s <= pz)

    lo0 = jnp.min(key, axis=1, keepdims=True) - jnp.int32(1)
    hi0 = jnp.max(key, axis=1, keepdims=True)

    def bis_body(_, lohi):
        lo, hi = lohi
        mid = (lo >> 1) + (hi >> 1) + (lo & hi & jnp.int32(1))
        kp = keep(mid)
        return jnp.where(kp, lo, mid), jnp.where(kp, mid, hi)

    _, ustar = lax.fori_loop(0, 32, bis_body, (lo0, hi0))

    gt = key > ustar
    eq = key == ustar
    gti = gt.astype(jnp.int32)
    eqi = eq.astype(jnp.int32)
    c_gt = jnp.sum(gti, axis=1, keepdims=True)
    s_gt = jnp.sum(jnp.where(gt, probs, 0.0), axis=1, keepdims=True)
    n_eq = jnp.sum(eqi, axis=1, keepdims=True)
    q_t = jnp.max(jnp.where(eq, probs, 0.0), axis=1, keepdims=True)

    r_k = k - c_gt
    r_p = jnp.where(
        q_t > 0.0,
        jnp.floor((pz - s_gt) / q_t).astype(jnp.int32) + jnp.int32(1),
        k)
    r = jnp.minimum(jnp.minimum(r_k, r_p), n_eq)

    iota = lax.broadcasted_iota(jnp.int32, (ROWS, V), 1)

    def idx_body(_, lohi):
        lo, hi = lohi
        mid = (lo + hi) >> 1
        cnt = jnp.sum(eqi * (iota <= mid).astype(jnp.int32),
                      axis=1, keepdims=True)
        ge = cnt >= r
        return jnp.where(ge, lo, mid), jnp.where(ge, mid, hi)

    ilo0 = jnp.full((ROWS, 1), -1, jnp.int32)
    ihi0 = jnp.full((ROWS, 1), V - 1, jnp.int32)
    _, istar = lax.fori_loop(0, 17, idx_body, (ilo0, ihi0))

    kept = gt | (eq & (iota <= istar))
    d = s_gt + r.astype(jnp.float32) * q_t
    out_ref[...] = jnp.where(kept, probs / d, 0.0)


def kernel(logits, top_ps, top_ks):
    b = logits.shape[0]
    ps = top_ps.astype(jnp.float32).reshape(b, 1)
    ks = top_ks.astype(jnp.int32).reshape(b, 1)
    return pl.pallas_call(
        _block,
        grid=(b // ROWS,),
        in_specs=[
            pl.BlockSpec((ROWS, V), lambda i: (i, 0)),
            pl.BlockSpec((ROWS, 1), lambda i: (i, 0)),
            pl.BlockSpec((ROWS, 1), lambda i: (i, 0)),
        ],
        out_specs=pl.BlockSpec((ROWS, V), lambda i: (i, 0)),
        out_shape=jax.ShapeDtypeStruct((b, V), jnp.float32),
    )(logits, ps, ks)


# speedup vs baseline: 16.8679x; 1.0226x over previous
import functools

import jax
import jax.numpy as jnp
from jax import lax
from jax.experimental import pallas as pl
from jax.experimental.pallas import tpu as pltpu
from jax.experimental.pallas import tpu_sc as plsc

B = 128
V = 100000
ROWS = 8
NC = 2
NS = 16
NW = NC * NS
RPW = B // NW
NV = V // 16
NBUCKET = 2048
CAP = 8192
KMIN = 1000
I32MIN = -2147483648
MASK31 = 0x7FFFFFFF


def _to_key(bits):
    return jnp.where(bits >= 0, bits,
                     jnp.int32(-1) - (bits & jnp.int32(MASK31)))


def _from_key(key):
    bits = jnp.where(key >= 0, key,
                     (jnp.int32(-1) - key) | jnp.int32(I32MIN))
    return lax.bitcast_convert_type(bits, jnp.float32)


def _sc_select_body(logits_hbm, scalf_hbm, scali_hbm, outf_hbm, outi_hbm,
                    row_v, hist_v, ckey_v, cidx_v, sif_v, sii_v,
                    sof_v, soi_v):
    wid = lax.axis_index("s") * NC + lax.axis_index("c")
    lane = lax.iota(jnp.int32, 16)

    for rr in range(RPW):
        row = wid * RPW + rr
        pltpu.sync_copy(logits_hbm.at[row], row_v)
        pltpu.sync_copy(scalf_hbm.at[row], sif_v)
        pltpu.sync_copy(scali_hbm.at[row], sii_v)
        p = sif_v[pl.ds(0, 16)][0]
        k = sii_v[pl.ds(0, 16)][0]

        def zero_body(j, _):
            hist_v[pl.ds(j * 16, 16)] = jnp.zeros((16,), jnp.int32)
            return 0
        lax.fori_loop(0, NBUCKET // 16, zero_body, 0)

        ones = jnp.ones((16,), jnp.int32)

        def scan1_body(i, mv):
            v = row_v[pl.ds(i * 16, 16)]
            bits = lax.bitcast_convert_type(v, jnp.int32)
            key = _to_key(bits)
            bucket = (key >> 21) + jnp.int32(NBUCKET // 2)
            plsc.addupdate_scatter(hist_v, [bucket], ones)
            return jnp.maximum(mv, v)

        mv = lax.fori_loop(0, NV, scan1_body,
                           jnp.full((16,), -jnp.inf, jnp.float32))
        m = jnp.max(mv)

        def tau_body(j, carry):
            acc, best = carry
            j2 = NBUCKET // 16 - 1 - j
            h = hist_v[pl.ds(j2 * 16, 16)]
            hr = lax.rev(h, (0,))
            cs = plsc.cumsum(hr) + acc
            crossed = cs >= KMIN
            has = jnp.any(crossed)
            ffs = jnp.max(plsc.all_reduce_ffs(crossed))
            bucket_c = j2 * 16 + 15 - ffs
            best = jnp.where((best < 0) & has, bucket_c, best)
            return acc + jnp.sum(h), best

        _, best = lax.fori_loop(0, NBUCKET // 16, tau_body,
                                (jnp.int32(0), jnp.int32(-1)))
        tau_key = (best - jnp.int32(NBUCKET // 2)) << 21

        def scan2_body(i, carry):
            zv, off = carry
            v = row_v[pl.ds(i * 16, 16)]
            zv = zv + jnp.exp(v - m)
            bits = lax.bitcast_convert_type(v, jnp.int32)
            key = _to_key(bits)
            msk = (key >= tau_key) & (off <= CAP - 16)
            idx16 = lane + i * 16
            plsc.store_compressed(ckey_v.at[pl.ds(off, 16)], key, mask=msk)
            plsc.store_compressed(cidx_v.at[pl.ds(off, 16)], idx16, mask=msk)
            off = off + jnp.max(plsc.all_reduce_population_count(msk))
            return zv, off

        zv, ncand = lax.fori_loop(0, NV, scan2_body,
                                  (jnp.zeros((16,), jnp.float32),
                                   jnp.int32(0)))
        z = jnp.sum(zv)
        pz = p * z
        km1 = k - jnp.int32(1)
        ncv = (ncand + jnp.int32(15)) >> 4

        def cs_scan(mid):
            def body(j, carry):
                cnt, sv = carry
                base = j * 16
                ki = ckey_v[pl.ds(base, 16)]
                valid = (lane + base) < ncand
                gt = (ki > mid) & valid
                cnt = cnt + jnp.where(gt, 1, 0)
                sv = sv + jnp.where(gt, jnp.exp(_from_key(ki) - m), 0.0)
                return cnt, sv
            cnt, sv = lax.fori_loop(0, ncv, body,
                                    (jnp.zeros((16,), jnp.int32),
                                     jnp.zeros((16,), jnp.float32)))
            return jnp.sum(cnt), jnp.sum(sv)

        def bis_body(_, lohi):
            lo, hi = lohi
            mid = (lo >> 1) + (hi >> 1) + (lo & hi & jnp.int32(1))
            c, s = cs_scan(mid)
            kp = (c <= km1) & (s <= pz)
            return jnp.where(kp, lo, mid), jnp.where(kp, mid, hi)

        hi0 = _to_key(lax.bitcast_convert_type(m, jnp.int32))
        _, ustar = lax.fori_loop(0, 32, bis_body,
                                 (tau_key - jnp.int32(1), hi0))

        c_gt, s_gt = cs_scan(ustar)
        qv = jnp.exp(jnp.broadcast_to(_from_key(ustar) - m, (16,)))
        r_k = k - c_gt
        ratio_v = jnp.minimum((pz - s_gt) / qv, 1e6)
        r_p = jnp.max(ratio_v.astype(jnp.int32)) + jnp.int32(1)

        def neq_body(j, acc):
            base = j * 16
            ki = ckey_v[pl.ds(base, 16)]
            eq = (ki == ustar) & ((lane + base) < ncand)
            return acc + jnp.sum(jnp.where(eq, 1, 0))

        n_eq = lax.fori_loop(0, ncv, neq_body, jnp.int32(0))
        r = jnp.minimum(jnp.minimum(r_k, r_p), n_eq)

        def tie_body(j, carry):
            cnt, istar = carry
            base = j * 16
            ki = ckey_v[pl.ds(base, 16)]
            ix = cidx_v[pl.ds(base, 16)]
            eq = (ki == ustar) & ((lane + base) < ncand)
            eqi = jnp.where(eq, 1, 0)
            ranks = plsc.cumsum(eqi)
            sel = eq & ((cnt + ranks) <= r)
            istar = jnp.maximum(istar, jnp.max(jnp.where(sel, ix, -1)))
            return cnt + jnp.sum(eqi), istar

        _, istar = lax.fori_loop(0, ncv, tie_body,
                                 (jnp.int32(0), jnp.int32(-1)))

        inv_dv = 1.0 / (s_gt + r.astype(jnp.float32) * qv)

        sof_v[...] = jnp.where(lane == 0, m,
                               jnp.where(lane == 1, inv_dv, 0.0))
        soi_v[...] = jnp.where(lane == 0, ustar,
                               jnp.where(lane == 1, istar, 0))
        pltpu.sync_copy(sof_v, outf_hbm.at[row])
        pltpu.sync_copy(soi_v, outi_hbm.at[row])


_sc_select = functools.partial(
    pl.kernel,
    out_type=[
        jax.ShapeDtypeStruct((B, 16), jnp.float32),
        jax.ShapeDtypeStruct((B, 16), jnp.int32),
    ],
    mesh=plsc.VectorSubcoreMesh(core_axis_name="c", subcore_axis_name="s"),
    compiler_params=pltpu.CompilerParams(needs_layout_passes=False),
    scratch_types=[
        pltpu.VMEM((V,), jnp.float32),
        pltpu.VMEM((NBUCKET,), jnp.int32),
        pltpu.VMEM((CAP,), jnp.int32),
        pltpu.VMEM((CAP,), jnp.int32),
        pltpu.VMEM((16,), jnp.float32),
        pltpu.VMEM((16,), jnp.int32),
        pltpu.VMEM((16,), jnp.float32),
        pltpu.VMEM((16,), jnp.int32),
    ],
)(_sc_select_body)


def _tc_out_block(logits_ref, f_ref, i_ref, out_ref):
    x = logits_ref[...]
    m = f_ref[:, 0:1]
    inv_d = f_ref[:, 1:2]
    ustar = i_ref[:, 0:1]
    istar = i_ref[:, 1:2]
    bits = lax.bitcast_convert_type(x, jnp.int32)
    key = _to_key(bits)
    iota = lax.broadcasted_iota(jnp.int32, (ROWS, V), 1)
    kept = (key > ustar) | ((key == ustar) & (iota <= istar))
    out_ref[...] = jnp.where(kept, jnp.exp(x - m) * inv_d, 0.0)


def kernel(logits, top_ps, top_ks):
    lane = jnp.arange(16)
    scalf = jnp.where(lane[None, :] == 0,
                      top_ps.astype(jnp.float32)[:, None], 0.0)
    scali = jnp.where(lane[None, :] == 0,
                      top_ks.astype(jnp.int32)[:, None], 0)
    outf, outi = _sc_select(logits, scalf, scali)
    return pl.pallas_call(
        _tc_out_block,
        grid=(B // ROWS,),
        in_specs=[
            pl.BlockSpec((ROWS, V), lambda i: (i, 0)),
            pl.BlockSpec((ROWS, 16), lambda i: (i, 0)),
            pl.BlockSpec((ROWS, 16), lambda i: (i, 0)),
        ],
        out_specs=pl.BlockSpec((ROWS, V), lambda i: (i, 0)),
        out_shape=jax.ShapeDtypeStruct((B, V), jnp.float32),
    )(logits, outf, outi)


# speedup vs baseline: 20.8677x; 1.2371x over previous
import functools

import jax
import jax.numpy as jnp
from jax import lax
from jax.experimental import pallas as pl
from jax.experimental.pallas import tpu as pltpu
from jax.experimental.pallas import tpu_sc as plsc

B = 128
V = 100000
ROWS = 8
NC = 2
NS = 16
NW = NC * NS
RPW = B // NW
NV = V // 16
NBUCKET = 2048
CAP = 8192
KMIN = 1000
I32MIN = -2147483648
MASK31 = 0x7FFFFFFF


def _to_key(bits):
    return jnp.where(bits >= 0, bits,
                     jnp.int32(-1) - (bits & jnp.int32(MASK31)))


def _from_key(key):
    bits = jnp.where(key >= 0, key,
                     (jnp.int32(-1) - key) | jnp.int32(I32MIN))
    return lax.bitcast_convert_type(bits, jnp.float32)


U1 = 10
U2 = 4


def _sc_select_body(logits_hbm, scalf_hbm, scali_hbm, outf_hbm, outi_hbm,
                    row_v, hist_v, ckey_v, cprob_v, cidx_v, sif_v, sii_v,
                    sof_v, soi_v):
    wid = lax.axis_index("s") * NC + lax.axis_index("c")
    lane = lax.iota(jnp.int32, 16)
    rbase = wid * RPW

    pltpu.sync_copy(scalf_hbm.at[pl.ds(rbase, RPW)], sif_v)
    pltpu.sync_copy(scali_hbm.at[pl.ds(rbase, RPW)], sii_v)

    for rr in range(RPW):
        row = rbase + rr
        pltpu.sync_copy(logits_hbm.at[row], row_v)
        p = sif_v[rr, pl.ds(0, 16)][0]
        k = sii_v[rr, pl.ds(0, 16)][0]

        def zero_body(j, _):
            hist_v[pl.ds(j * 16, 16)] = jnp.zeros((16,), jnp.int32)
            return 0
        lax.fori_loop(0, NBUCKET // 16, zero_body, 0)

        ones = jnp.ones((16,), jnp.int32)

        def scan1_body(i, mv):
            for u in range(U1):
                v = row_v[pl.ds((i * U1 + u) * 16, 16)]
                bits = lax.bitcast_convert_type(v, jnp.int32)
                key = _to_key(bits)
                bucket = (key >> 21) + jnp.int32(NBUCKET // 2)
                plsc.addupdate_scatter(hist_v, [bucket], ones)
                mv = jnp.maximum(mv, v)
            return mv

        mv = lax.fori_loop(0, NV // U1, scan1_body,
                           jnp.full((16,), -jnp.inf, jnp.float32))
        m = jnp.max(mv)

        def tau_body(j, carry):
            acc, best = carry
            j2 = NBUCKET // 16 - 1 - j
            h = hist_v[pl.ds(j2 * 16, 16)]
            hr = lax.rev(h, (0,))
            cs = plsc.cumsum(hr) + acc
            crossed = cs >= KMIN
            has = jnp.any(crossed)
            ffs = plsc.all_reduce_ffs(crossed)[0]
            bucket_c = j2 * 16 + 15 - ffs
            best = jnp.where((best < 0) & has, bucket_c, best)
            return cs[15], best

        _, best = lax.fori_loop(0, NBUCKET // 16, tau_body,
                                (jnp.int32(0), jnp.int32(-1)))
        tau_key = (best - jnp.int32(NBUCKET // 2)) << 21

        def scan2_body(i, carry):
            zv, off = carry
            for u in range(U1):
                iu = i * U1 + u
                v = row_v[pl.ds(iu * 16, 16)]
                zv = zv + jnp.exp(v - m)
                bits = lax.bitcast_convert_type(v, jnp.int32)
                key = _to_key(bits)
                msk = (key >= tau_key) & (off <= CAP - 16)
                idx16 = lane + iu * 16
                plsc.store_compressed(ckey_v.at[pl.ds(off, 16)], key,
                                      mask=msk)
                plsc.store_compressed(cidx_v.at[pl.ds(off, 16)], idx16,
                                      mask=msk)
                off = off + plsc.all_reduce_population_count(msk)[0]
            return zv, off

        zv, ncand = lax.fori_loop(0, NV // U1, scan2_body,
                                  (jnp.zeros((16,), jnp.float32),
                                   jnp.int32(0)))
        z = jnp.sum(zv)
        pz = p * z
        km1 = k - jnp.int32(1)
        ng = (ncand + jnp.int32(16 * U2 - 1)) >> (4 + U2.bit_length() - 1)

        def prob_body(j, _):
            for u in range(U2):
                base = (j * U2 + u) * 16
                ki = ckey_v[pl.ds(base, 16)]
                cprob_v[pl.ds(base, 16)] = jnp.exp(_from_key(ki) - m)
            return 0
        lax.fori_loop(0, ng, prob_body, 0)

        def cs_scan(mid):
            def body(j, carry):
                cnt, sv = carry
                for u in range(U2):
                    base = (j * U2 + u) * 16
                    ki = ckey_v[pl.ds(base, 16)]
                    pv = cprob_v[pl.ds(base, 16)]
                    gt = (ki > mid) & ((lane + base) < ncand)
                    cnt = cnt + jnp.where(gt, 1, 0)
                    sv = sv + jnp.where(gt, pv, 0.0)
                return cnt, sv
            cnt, sv = lax.fori_loop(0, ng, body,
                                    (jnp.zeros((16,), jnp.int32),
                                     jnp.zeros((16,), jnp.float32)))
            return jnp.sum(cnt), jnp.sum(sv)

        def bis_cond(state):
            it, lo, hi = state
            return (it < 32) & (hi > lo + 1)

        def bis_body(state):
            it, lo, hi = state
            mid = (lo >> 1) + (hi >> 1) + (lo & hi & jnp.int32(1))
            c, s = cs_scan(mid)
            kp = (c <= km1) & (s <= pz)
            return (it + 1, jnp.where(kp, lo, mid), jnp.where(kp, mid, hi))

        hi0 = _to_key(lax.bitcast_convert_type(m, jnp.int32))
        _, _, ustar = lax.while_loop(bis_cond, bis_body,
                                     (jnp.int32(0), tau_key - jnp.int32(1),
                                      hi0))

        def stats_body(j, carry):
            cnt, sv, ne = carry
            for u in range(U2):
                base = (j * U2 + u) * 16
                ki = ckey_v[pl.ds(base, 16)]
                pv = cprob_v[pl.ds(base, 16)]
                valid = (lane + base) < ncand
                gt = (ki > ustar) & valid
                eq = (ki == ustar) & valid
                cnt = cnt + jnp.where(gt, 1, 0)
                sv = sv + jnp.where(gt, pv, 0.0)
                ne = ne + jnp.where(eq, 1, 0)
            return cnt, sv, ne

        cntv, sgv, nev = lax.fori_loop(
            0, ng, stats_body,
            (jnp.zeros((16,), jnp.int32), jnp.zeros((16,), jnp.float32),
             jnp.zeros((16,), jnp.int32)))
        c_gt = jnp.sum(cntv)
        s_gt = jnp.sum(sgv)
        n_eq = jnp.sum(nev)

        qv = jnp.exp(jnp.broadcast_to(_from_key(ustar) - m, (16,)))
        r_k = k - c_gt
        ratio_v = jnp.minimum((pz - s_gt) / qv, 1e6)
        r_p = ratio_v.astype(jnp.int32)[0] + jnp.int32(1)
        r = jnp.minimum(jnp.minimum(r_k, r_p), n_eq)

        def tie_body(j, carry):
            cnt, istar = carry
            base = j * 16
            ki = ckey_v[pl.ds(base, 16)]
            ix = cidx_v[pl.ds(base, 16)]
            eq = (ki == ustar) & ((lane + base) < ncand)
            eqi = jnp.where(eq, 1, 0)
            ranks = plsc.cumsum(eqi)
            sel = eq & ((cnt + ranks) <= r)
            istar = jnp.maximum(istar, jnp.max(jnp.where(sel, ix, -1)))
            return cnt + ranks[15], istar

        ncv = (ncand + jnp.int32(15)) >> 4
        _, istar = lax.fori_loop(0, ncv, tie_body,
                                 (jnp.int32(0), jnp.int32(-1)))

        inv_dv = 1.0 / (s_gt + r.astype(jnp.float32) * qv)

        sof_v[rr, pl.ds(0, 16)] = jnp.where(lane == 0, m,
                                            jnp.where(lane == 1, inv_dv,
                                                      0.0))
        soi_v[rr, pl.ds(0, 16)] = jnp.where(lane == 0, ustar,
                                            jnp.where(lane == 1, istar, 0))

    pltpu.sync_copy(sof_v, outf_hbm.at[pl.ds(rbase, RPW)])
    pltpu.sync_copy(soi_v, outi_hbm.at[pl.ds(rbase, RPW)])


_sc_select = functools.partial(
    pl.kernel,
    out_type=[
        jax.ShapeDtypeStruct((B, 16), jnp.float32),
        jax.ShapeDtypeStruct((B, 16), jnp.int32),
    ],
    mesh=plsc.VectorSubcoreMesh(core_axis_name="c", subcore_axis_name="s"),
    compiler_params=pltpu.CompilerParams(needs_layout_passes=False),
    scratch_types=[
        pltpu.VMEM((V,), jnp.float32),
        pltpu.VMEM((NBUCKET,), jnp.int32),
        pltpu.VMEM((CAP + 64,), jnp.int32),
        pltpu.VMEM((CAP + 64,), jnp.float32),
        pltpu.VMEM((CAP + 64,), jnp.int32),
        pltpu.VMEM((RPW, 16), jnp.float32),
        pltpu.VMEM((RPW, 16), jnp.int32),
        pltpu.VMEM((RPW, 16), jnp.float32),
        pltpu.VMEM((RPW, 16), jnp.int32),
    ],
)(_sc_select_body)


def _tc_out_block(logits_ref, f_ref, i_ref, out_ref):
    x = logits_ref[...]
    m = f_ref[:, 0:1]
    inv_d = f_ref[:, 1:2]
    ustar = i_ref[:, 0:1]
    istar = i_ref[:, 1:2]
    bits = lax.bitcast_convert_type(x, jnp.int32)
    key = _to_key(bits)
    iota = lax.broadcasted_iota(jnp.int32, (ROWS, V), 1)
    kept = (key > ustar) | ((key == ustar) & (iota <= istar))
    out_ref[...] = jnp.where(kept, jnp.exp(x - m) * inv_d, 0.0)


def kernel(logits, top_ps, top_ks):
    lane = jnp.arange(16)
    scalf = jnp.where(lane[None, :] == 0,
                      top_ps.astype(jnp.float32)[:, None], 0.0)
    scali = jnp.where(lane[None, :] == 0,
                      top_ks.astype(jnp.int32)[:, None], 0)
    outf, outi = _sc_select(logits, scalf, scali)
    return pl.pallas_call(
        _tc_out_block,
        grid=(B // ROWS,),
        in_specs=[
            pl.BlockSpec((ROWS, V), lambda i: (i, 0)),
            pl.BlockSpec((ROWS, 16), lambda i: (i, 0)),
            pl.BlockSpec((ROWS, 16), lambda i: (i, 0)),
        ],
        out_specs=pl.BlockSpec((ROWS, V), lambda i: (i, 0)),
        out_shape=jax.ShapeDtypeStruct((B, V), jnp.float32),
    )(logits, outf, outi)
